# Initial kernel scaffold; baseline (speedup 1.0000x reference)
import jax
import jax.numpy as jnp
from jax.experimental import pallas as pl


def kernel(node_features, adj_indices, adj_values, W, b):
    raise NotImplementedError("write your pallas kernel here")


# speedup vs baseline: 4.4533x; 4.4533x over previous
import functools

import jax
import jax.numpy as jnp
from jax import lax
from jax.experimental import pallas as pl
from jax.experimental.pallas import tpu as pltpu
from jax.experimental.pallas import tpu_sc as plsc


def _mm_body(x_ref, w_ref, o_ref):
    o_ref[...] = jnp.dot(x_ref[...], w_ref[...],
                         preferred_element_type=jnp.float32)


def _matmul(x, w):
    n, d_in = x.shape
    d_out = w.shape[1]
    blk = 1000
    return pl.pallas_call(
        _mm_body,
        grid=(n // blk,),
        in_specs=[pl.BlockSpec((blk, d_in), lambda i: (i, 0)),
                  pl.BlockSpec((d_in, d_out), lambda i: (0, 0))],
        out_specs=pl.BlockSpec((blk, d_out), lambda i: (i, 0)),
        out_shape=jax.ShapeDtypeStruct((n, d_out), jnp.float32),
    )(x, w)


def _comb_body(p_ref, b_ref, o_ref):
    o_ref[...] = p_ref[0] + p_ref[1] + b_ref[...]


def _combine(p, b2d):
    nc, n, d = p.shape
    blk = 1000
    return pl.pallas_call(
        _comb_body,
        grid=(n // blk,),
        in_specs=[pl.BlockSpec((nc, blk, d), lambda i: (0, i, 0)),
                  pl.BlockSpec((1, d), lambda i: (0, 0))],
        out_specs=pl.BlockSpec((blk, d), lambda i: (i, 0)),
        out_shape=jax.ShapeDtypeStruct((n, d), jnp.float32),
    )(p, b2d)


@functools.lru_cache(maxsize=None)
def _make_sc_spmm(n, d, e):
    info = plsc.get_sparse_core_info()
    nc, ns, nl = info.num_cores, info.num_subcores, info.num_lanes
    nw = nc * ns
    epw = e // nw
    chunk_e = 80
    n_chunks = epw // chunk_e
    sl_rows = 400
    n_slices = n // sl_rows
    sl_per_tile = -(-n_slices // ns)
    zrows = 80
    mesh = plsc.VectorSubcoreMesh(core_axis_name="c", subcore_axis_name="s")

    @functools.partial(
        pl.kernel,
        out_type=jax.ShapeDtypeStruct((nc, n, d), jnp.float32),
        mesh=mesh,
        scratch_types=[
            pltpu.VMEM((chunk_e,), jnp.int32),
            pltpu.VMEM((chunk_e,), jnp.int32),
            pltpu.VMEM((chunk_e,), jnp.float32),
            pltpu.VMEM((chunk_e, d), jnp.float32),
            pltpu.VMEM((zrows, d), jnp.float32),
            pltpu.VMEM_SHARED((n, d), jnp.float32),
            pltpu.SemaphoreType.DMA,
        ],
    )
    def spmm(support_hbm, rows_hbm, cols_hbm, vals_hbm, out_hbm,
             cidx_v, ridx_v, vals_v, gat_v, zbuf_v, acc_sh, sem):
        cid = lax.axis_index("c")
        sid = lax.axis_index("s")
        wid = sid * nc + cid

        zero16 = jnp.zeros((nl,), jnp.float32)

        def zrow(i, carry):
            for j in range(d // nl):
                zbuf_v[i, pl.ds(j * nl, nl)] = zero16
            return carry
        lax.fori_loop(0, zrows, zrow, 0)
        for k in range(sl_per_tile):
            sl_id = sid + ns * k
            @pl.when(sl_id < n_slices)
            def _():
                off = pl.multiple_of(sl_id * sl_rows, 8)
                for z in range(sl_rows // zrows):
                    pltpu.sync_copy(zbuf_v,
                                    acc_sh.at[pl.ds(off + z * zrows, zrows)])
        plsc.subcore_barrier()

        base = wid * epw

        def chunk(i, carry):
            off = base + i * chunk_e
            pltpu.sync_copy(cols_hbm.at[pl.ds(off, chunk_e)], cidx_v)
            pltpu.sync_copy(vals_hbm.at[pl.ds(off, chunk_e)], vals_v)
            pltpu.sync_copy(rows_hbm.at[pl.ds(off, chunk_e)], ridx_v)
            pltpu.async_copy(support_hbm.at[cidx_v], gat_v, sem).wait()

            def scale(g, c2):
                vblock = vals_v[pl.ds(g * nl, nl)]
                base_e = g * nl
                for k in range(nl):
                    v = vblock[k]
                    ei = base_e + k
                    for j in range(d // nl):
                        sl = pl.ds(j * nl, nl)
                        gat_v[ei, sl] = gat_v[ei, sl] * v
                return c2
            lax.fori_loop(0, chunk_e // nl, scale, 0)
            pltpu.sync_copy(gat_v, acc_sh.at[ridx_v], add=True)
            return carry
        lax.fori_loop(0, n_chunks, chunk, 0)

        plsc.subcore_barrier()
        for k in range(sl_per_tile):
            sl_id = sid + ns * k
            @pl.when(sl_id < n_slices)
            def _():
                off = pl.multiple_of(sl_id * sl_rows, 8)
                pltpu.sync_copy(acc_sh.at[pl.ds(off, sl_rows)],
                                out_hbm.at[cid, pl.ds(off, sl_rows)])

    return spmm


def kernel(node_features, adj_indices, adj_values, W, b):
    n, _ = node_features.shape
    d = W.shape[1]
    e = adj_values.shape[0]
    support = _matmul(node_features, W)
    spmm = _make_sc_spmm(n, d, e)
    partials = spmm(support, adj_indices[0], adj_indices[1], adj_values)
    return _combine(partials, b.reshape(1, d))
